# Initial kernel scaffold; baseline (speedup 1.0000x reference)
import jax
import jax.numpy as jnp
from jax.experimental import pallas as pl


def kernel(x, T, d_ew, d_edges, d_dist, W, b):
    raise NotImplementedError("write your pallas kernel here")


# speedup vs baseline: 21.5555x; 21.5555x over previous
import functools

import jax
import jax.numpy as jnp
from jax import lax
from jax.experimental import pallas as pl
from jax.experimental.pallas import tpu as pltpu

_ALPHA = 0.2
_EB = 512
_NPAD = 1024


def _body(nblk, ni_ref, nj_ref, w0_ref, w1_ref, dd_ref, xT_ref, Wc_ref,
          bt_ref, out_ref, acc_ref):
    i = pl.program_id(0)

    @pl.when(i == 0)
    def _():
        acc_ref[...] = jnp.zeros_like(acc_ref)

    ni = ni_ref[0, 0, :]
    nj = nj_ref[0, 0, :]
    w0 = w0_ref[0, 0, :]
    w1 = w1_ref[0, 0, :]
    dd = dd_ref[0, 0, :]

    iota_n = lax.broadcasted_iota(jnp.int32, (_EB, _NPAD), 1)
    oh_i = (iota_n == ni[:, None]).astype(jnp.float32)
    xg = jnp.dot(oh_i, xT_ref[...], preferred_element_type=jnp.float32)

    lane = lax.broadcasted_iota(jnp.int32, (_EB, 128), 1)
    xdup = jnp.concatenate(
        [xg, xg, jnp.zeros((_EB, 64), jnp.float32)], axis=1)
    wb = jnp.where(lane < 32, w0[:, None], w1[:, None])
    msgs = jnp.where(lane < 64, wb * xdup, 0.0)
    msgs = jnp.where(lane == 64, (dd * w0)[:, None], msgs)
    msgs = jnp.where(lane == 65, (dd * w1)[:, None], msgs)

    iota_m = lax.broadcasted_iota(jnp.int32, (_NPAD, _EB), 0)
    oh_jT = (iota_m == nj[None, :]).astype(jnp.float32)
    acc_ref[...] += jnp.dot(oh_jT, msgs, preferred_element_type=jnp.float32)

    @pl.when(i == nblk - 1)
    def _():
        acc = acc_ref[...]
        out64 = jnp.dot(acc, Wc_ref[...],
                        preferred_element_type=jnp.float32) + bt_ref[0:1, :]
        parts = []
        for h in range(2):
            for t in range(4):
                base = 32 * h + 8 * t
                cur = out64[:, base:base + 8]
                if t == 0:
                    parts.append(cur)
                else:
                    prev = out64[:, base - 8:base]
                    parts.append(_ALPHA * prev + (1.0 - _ALPHA) * cur)
        sm = jnp.concatenate(parts, axis=1)
        out_ref[...] = 1.0 / (1.0 + jnp.exp(-sm))


def kernel(x, T, d_ew, d_edges, d_dist, W, b):
    del T
    _, T_, N, Cx = x.shape
    E = d_edges.shape[0]
    nblk = (E + _EB - 1) // _EB
    e_pad = nblk * _EB

    xT = x[0].transpose(1, 0, 2).reshape(N, T_ * Cx)
    xTp = jnp.zeros((_NPAD, T_ * Cx), jnp.float32).at[:N].set(xT)

    ni = jnp.zeros((e_pad,), jnp.int32).at[:E].set(d_edges[:, 0])
    nj = jnp.full((e_pad,), _NPAD - 8, jnp.int32).at[:E].set(d_edges[:, 1])
    w0 = jnp.zeros((e_pad,), jnp.float32).at[:E].set(d_ew[:, 0])
    w1 = jnp.zeros((e_pad,), jnp.float32).at[:E].set(d_ew[:, 1])
    dd = jnp.zeros((e_pad,), jnp.float32).at[:E].set(d_dist)

    ni3 = ni.reshape(nblk, 1, _EB)
    nj3 = nj.reshape(nblk, 1, _EB)
    w03 = w0.reshape(nblk, 1, _EB)
    w13 = w1.reshape(nblk, 1, _EB)
    dd3 = dd.reshape(nblk, 1, _EB)

    Wc = jnp.zeros((128, 64), jnp.float32)
    for h in range(2):
        for t in range(4):
            base = 32 * h + 8 * t
            Wc = Wc.at[base:base + 8, base:base + 8].set(W[:8, :])
    Wc = Wc.at[64, 0:32].set(jnp.tile(W[8, :], 4))
    Wc = Wc.at[65, 32:64].set(jnp.tile(W[8, :], 4))
    bt = jnp.broadcast_to(jnp.tile(b, 8)[None, :], (8, 64))

    eb_spec = pl.BlockSpec((1, 1, _EB), lambda i: (i, 0, 0))
    out = pl.pallas_call(
        functools.partial(_body, nblk),
        grid=(nblk,),
        in_specs=[
            eb_spec, eb_spec, eb_spec, eb_spec, eb_spec,
            pl.BlockSpec((_NPAD, T_ * Cx), lambda i: (0, 0)),
            pl.BlockSpec((128, 64), lambda i: (0, 0)),
            pl.BlockSpec((8, 64), lambda i: (0, 0)),
        ],
        out_specs=pl.BlockSpec((_NPAD, 64), lambda i: (0, 0)),
        out_shape=jax.ShapeDtypeStruct((_NPAD, 64), jnp.float32),
        scratch_shapes=[pltpu.VMEM((_NPAD, 128), jnp.float32)],
    )(ni3, nj3, w03, w13, dd3, xTp, Wc, bt)

    res = out[:N].reshape(N, 2, T_, 8).transpose(2, 0, 1, 3)
    return res[None]
